# Initial kernel scaffold; baseline (speedup 1.0000x reference)
import jax
import jax.numpy as jnp
from jax.experimental import pallas as pl


def kernel(item, item_counters):
    raise NotImplementedError("write your pallas kernel here")


# speedup vs baseline: 259.4177x; 259.4177x over previous
import functools

import jax
import jax.numpy as jnp
from jax import lax
from jax.experimental import pallas as pl
from jax.experimental.pallas import tpu as pltpu
from jax.experimental.pallas import tpu_sc as plsc

NUM_ITEMS = 1_000_000
N = 16384 * 200
CHUNK = 10_240

NC = 2
NS = 16
NW = NC * NS

PER_TILE_SCATTER = N // NS
SCATTER_CHUNKS = PER_TILE_SCATTER // CHUNK
PER_W_GATHER = N // NW
GATHER_CHUNKS = PER_W_GATHER // CHUNK

INIT_CHUNK = 62_496
INIT_BOUNCE = INIT_CHUNK // 4
INIT_TAIL = NUM_ITEMS - NS * INIT_CHUNK


def _body(item_ref, counters_ref, ones_ref, out_ref,
          table, idx_v, val_v, out_v, bounce_v, t16, c16):
    cid = lax.axis_index("c")
    sid = lax.axis_index("s")

    islot = (sid + cid * (NS // 2)) % NS
    ibase = islot * INIT_CHUNK

    def init_body(j, carry):
        off = ibase + j * INIT_BOUNCE
        pltpu.sync_copy(counters_ref.at[pl.ds(off, INIT_BOUNCE)], bounce_v)
        pltpu.sync_copy(bounce_v, table.at[pl.ds(off, INIT_BOUNCE)])
        return carry

    lax.fori_loop(0, INIT_CHUNK // INIT_BOUNCE, init_body, 0)

    @pl.when(sid == 0)
    def _init_tail():
        pltpu.sync_copy(counters_ref.at[pl.ds(NS * INIT_CHUNK, INIT_TAIL)],
                        bounce_v.at[pl.ds(0, INIT_TAIL)])
        pltpu.sync_copy(bounce_v.at[pl.ds(0, INIT_TAIL)],
                        table.at[pl.ds(NS * INIT_CHUNK, INIT_TAIL)])

    pltpu.sync_copy(ones_ref, val_v)
    plsc.subcore_barrier()

    def scatter_body(i, carry):
        slot = (i + cid * (SCATTER_CHUNKS // 2)) % SCATTER_CHUNKS
        base = sid * PER_TILE_SCATTER + slot * CHUNK
        pltpu.sync_copy(item_ref.at[pl.ds(base, CHUNK)], idx_v)
        pltpu.sync_copy(val_v, table.at[idx_v], add=True)
        return carry

    lax.fori_loop(0, SCATTER_CHUNKS, scatter_body, 0)
    plsc.subcore_barrier()

    @pl.when(sid == 0)
    def _fix_zero():
        pltpu.sync_copy(table.at[pl.ds(0, 16)], t16)
        pltpu.sync_copy(counters_ref.at[pl.ds(0, 16)], c16)
        lane = lax.iota(jnp.int32, 16)
        t16[...] = jnp.where(lane == 0, c16[...], t16[...])
        pltpu.sync_copy(t16, table.at[pl.ds(0, 16)])

    plsc.subcore_barrier()

    wid = sid * NC + cid

    def gather_body(i, carry):
        base = wid * PER_W_GATHER + i * CHUNK
        pltpu.sync_copy(item_ref.at[pl.ds(base, CHUNK)], idx_v)
        pltpu.sync_copy(table.at[idx_v], out_v)
        pltpu.sync_copy(out_v, out_ref.at[pl.ds(base, CHUNK)])
        return carry

    lax.fori_loop(0, GATHER_CHUNKS, gather_body, 0)


@jax.jit
def kernel(item, item_counters):
    mesh = plsc.VectorSubcoreMesh(core_axis_name="c", subcore_axis_name="s",
                                  num_cores=NC, num_subcores=NS)
    run = pl.kernel(
        _body,
        out_type=jax.ShapeDtypeStruct((N,), jnp.float32),
        mesh=mesh,
        scratch_types=[
            pltpu.VMEM_SHARED((NUM_ITEMS,), jnp.float32),
            pltpu.VMEM((CHUNK,), jnp.int32),
            pltpu.VMEM((CHUNK,), jnp.float32),
            pltpu.VMEM((CHUNK,), jnp.float32),
            pltpu.VMEM((INIT_BOUNCE,), jnp.float32),
            pltpu.VMEM((16,), jnp.float32),
            pltpu.VMEM((16,), jnp.float32),
        ],
    )
    ones = jnp.ones((CHUNK,), jnp.float32)
    out = run(item.reshape(-1), item_counters, ones)
    return out.reshape(item.shape)


# speedup vs baseline: 304.0749x; 1.1721x over previous
import jax
import jax.numpy as jnp
from jax import lax
from jax.experimental import pallas as pl
from jax.experimental.pallas import tpu as pltpu
from jax.experimental.pallas import tpu_sc as plsc

NUM_ITEMS = 1_000_000
N = 16384 * 200
CHUNK = 10_240

NC = 2
NS = 16
NW = NC * NS

PER_TILE_SCATTER = N // NS
SCATTER_CHUNKS = PER_TILE_SCATTER // CHUNK
PER_W_GATHER = N // NW
GATHER_CHUNKS = PER_W_GATHER // CHUNK

INIT_BOUNCE = 7_808
INIT_WIN = 8
INIT_CHUNK = INIT_BOUNCE * INIT_WIN
INIT_TAIL = NUM_ITEMS - NS * INIT_CHUNK


def _body(item_ref, counters_ref, ones_ref, out_ref, table,
          idx0, idx1, val_v, out0, out1, bnc0, bnc1, t16, c16,
          lsem0, lsem1, ssem0, ssem1, osem0, osem1, gsem0, gsem1):
    cid = lax.axis_index("c")
    sid = lax.axis_index("s")
    idxb, lsem = [idx0, idx1], [lsem0, lsem1]
    ssem = [ssem0, ssem1]
    outb, osem, gsem = [out0, out1], [osem0, osem1], [gsem0, gsem1]
    bncb = [bnc0, bnc1]

    vdesc = pltpu.async_copy(ones_ref, val_v, gsem0)
    islot = (sid + cid * (NS // 2)) % NS
    ibase = islot * INIT_CHUNK
    iloads = [None] * INIT_WIN
    istores = [None] * INIT_WIN
    iloads[0] = pltpu.async_copy(counters_ref.at[pl.ds(ibase, INIT_BOUNCE)],
                                 bnc0, lsem0)
    for j in range(INIT_WIN):
        iloads[j].wait()
        istores[j] = pltpu.async_copy(
            bncb[j % 2],
            table.at[pl.ds(ibase + j * INIT_BOUNCE, INIT_BOUNCE)],
            ssem[j % 2])
        if j >= 1:
            istores[j - 1].wait()
        if j + 1 < INIT_WIN:
            iloads[j + 1] = pltpu.async_copy(
                counters_ref.at[pl.ds(ibase + (j + 1) * INIT_BOUNCE,
                                      INIT_BOUNCE)],
                bncb[(j + 1) % 2], lsem[(j + 1) % 2])
    istores[INIT_WIN - 1].wait()

    @pl.when(sid == 0)
    def _init_tail():
        pltpu.sync_copy(counters_ref.at[pl.ds(NS * INIT_CHUNK, INIT_TAIL)],
                        bnc0.at[pl.ds(0, INIT_TAIL)])
        pltpu.sync_copy(bnc0.at[pl.ds(0, INIT_TAIL)],
                        table.at[pl.ds(NS * INIT_CHUNK, INIT_TAIL)])

    vdesc.wait()
    plsc.subcore_barrier()

    nsc = SCATTER_CHUNKS
    sbase = sid * PER_TILE_SCATTER

    def schunk(i):
        return sbase + ((i + cid * (nsc // 2)) % nsc) * CHUNK

    sloads = [None] * nsc
    sadds = [None] * nsc
    sloads[0] = pltpu.async_copy(item_ref.at[pl.ds(schunk(0), CHUNK)],
                                 idx0, lsem0)
    for i in range(nsc):
        sloads[i].wait()
        sadds[i] = pltpu.async_copy(val_v, table.at[idxb[i % 2]],
                                    ssem[i % 2], add=True)
        if i >= 1:
            sadds[i - 1].wait()
        if i + 1 < nsc:
            sloads[i + 1] = pltpu.async_copy(
                item_ref.at[pl.ds(schunk(i + 1), CHUNK)],
                idxb[(i + 1) % 2], lsem[(i + 1) % 2])
    sadds[nsc - 1].wait()

    wid = sid * NC + cid
    gb = wid * PER_W_GATHER
    ngc = GATHER_CHUNKS
    gloads = [None] * ngc
    for i in range(2):
        gloads[i] = pltpu.async_copy(
            item_ref.at[pl.ds(gb + i * CHUNK, CHUNK)], idxb[i], lsem[i])

    plsc.subcore_barrier()

    @pl.when(sid == 0)
    def _fix_zero():
        pltpu.sync_copy(table.at[pl.ds(0, 16)], t16)
        pltpu.sync_copy(counters_ref.at[pl.ds(0, 16)], c16)
        lane = lax.iota(jnp.int32, 16)
        t16[...] = jnp.where(lane == 0, c16[...], t16[...])
        pltpu.sync_copy(t16, table.at[pl.ds(0, 16)])

    plsc.subcore_barrier()

    gstores = [None] * ngc
    for i in range(ngc):
        gloads[i].wait()
        if i >= 2:
            gstores[i - 2].wait()
        g = pltpu.async_copy(table.at[idxb[i % 2]], outb[i % 2],
                             gsem[i % 2])
        g.wait()
        gstores[i] = pltpu.async_copy(
            outb[i % 2], out_ref.at[pl.ds(gb + i * CHUNK, CHUNK)],
            osem[i % 2])
        if i + 2 < ngc:
            gloads[i + 2] = pltpu.async_copy(
                item_ref.at[pl.ds(gb + (i + 2) * CHUNK, CHUNK)],
                idxb[i % 2], lsem[i % 2])
    gstores[ngc - 2].wait()
    gstores[ngc - 1].wait()


@jax.jit
def kernel(item, item_counters):
    mesh = plsc.VectorSubcoreMesh(core_axis_name="c", subcore_axis_name="s",
                                  num_cores=NC, num_subcores=NS)
    run = pl.kernel(
        _body,
        out_type=jax.ShapeDtypeStruct((N,), jnp.float32),
        mesh=mesh,
        scratch_types=[
            pltpu.VMEM_SHARED((NUM_ITEMS,), jnp.float32),
            pltpu.VMEM((CHUNK,), jnp.int32),
            pltpu.VMEM((CHUNK,), jnp.int32),
            pltpu.VMEM((CHUNK,), jnp.float32),
            pltpu.VMEM((CHUNK,), jnp.float32),
            pltpu.VMEM((CHUNK,), jnp.float32),
            pltpu.VMEM((INIT_BOUNCE,), jnp.float32),
            pltpu.VMEM((INIT_BOUNCE,), jnp.float32),
            pltpu.VMEM((16,), jnp.float32),
            pltpu.VMEM((16,), jnp.float32),
        ] + [pltpu.SemaphoreType.DMA] * 8,
    )
    ones = jnp.ones((CHUNK,), jnp.float32)
    out = run(item.reshape(-1), item_counters, ones)
    return out.reshape(item.shape)
